# Initial kernel scaffold; baseline (speedup 1.0000x reference)
import jax
import jax.numpy as jnp
from jax.experimental import pallas as pl


def kernel(x, edge_index, W1, b1, W2, b2):
    raise NotImplementedError("write your pallas kernel here")


# speedup vs baseline: 11.5088x; 11.5088x over previous
import functools

import jax
import jax.numpy as jnp
from jax import lax
from jax.experimental import pallas as pl
from jax.experimental.pallas import tpu as pltpu
from jax.experimental.pallas import tpu_sc as plsc

NC = 2
NS = 16
NW = NC * NS
CH = 128
DW = 128
BM = 256


def _sc_mesh():
    return plsc.VectorSubcoreMesh(
        core_axis_name="c", subcore_axis_name="s",
        num_cores=NC, num_subcores=NS)


def _deg_kernel(dst_hbm, ones_hbm, zeros_hbm, out_hbm, idx_v, ones_v, acc):
    cid = lax.axis_index("c")
    sid = lax.axis_index("s")
    wid = cid * NS + sid
    nchunk = idx_v.shape[0]
    rows_pt = acc.shape[0] // NS
    r0 = sid * rows_pt

    pltpu.sync_copy(zeros_hbm.at[pl.ds(r0, rows_pt)], acc.at[pl.ds(r0, rows_pt)])
    pltpu.sync_copy(ones_hbm, ones_v)
    pltpu.sync_copy(dst_hbm.at[wid], idx_v)
    plsc.subcore_barrier()

    def body(k, _):
        pltpu.sync_copy(ones_v, acc.at[idx_v.at[k]], add=True)
        return ()

    lax.fori_loop(0, nchunk, body, (), unroll=False)
    plsc.subcore_barrier()
    pltpu.sync_copy(acc.at[pl.ds(r0, rows_pt)], out_hbm.at[cid, pl.ds(r0, rows_pt)])


def _deg_call(dst_p, ones16, z16, n_acc, nchunk):
    kfn = functools.partial(
        pl.kernel,
        out_type=jax.ShapeDtypeStruct((NC, n_acc, DW), jnp.float32),
        mesh=_sc_mesh(),
        scratch_types=[
            pltpu.VMEM((nchunk, CH), jnp.int32),
            pltpu.VMEM((CH, DW), jnp.float32),
            pltpu.VMEM_SHARED((n_acc, DW), jnp.float32),
        ],
    )(_deg_kernel)
    return kfn(dst_p, ones16, z16)


def _agg_kernel(y_hbm, z_hbm, src_hbm, dst_hbm, out_hbm,
                idx_s, idx_d, rows_v, acc, sem):
    cid = lax.axis_index("c")
    sid = lax.axis_index("s")
    wid = cid * NS + sid
    nchunk = idx_s.shape[0]
    rows_pt = acc.shape[0] // NS
    r0 = sid * rows_pt

    @pl.when(cid == 0)
    def _():
        pltpu.sync_copy(y_hbm.at[pl.ds(r0, rows_pt)], acc.at[pl.ds(r0, rows_pt)])

    @pl.when(cid != 0)
    def _():
        pltpu.sync_copy(z_hbm.at[pl.ds(r0, rows_pt)], acc.at[pl.ds(r0, rows_pt)])

    pltpu.sync_copy(src_hbm.at[wid], idx_s)
    pltpu.sync_copy(dst_hbm.at[wid], idx_d)
    plsc.subcore_barrier()

    def body(k, _):
        pltpu.async_copy(y_hbm.at[idx_s.at[k]], rows_v, sem).wait()
        pltpu.sync_copy(rows_v, acc.at[idx_d.at[k]], add=True)
        return ()

    lax.fori_loop(0, nchunk, body, (), unroll=False)
    plsc.subcore_barrier()
    pltpu.sync_copy(acc.at[pl.ds(r0, rows_pt)], out_hbm.at[cid, pl.ds(r0, rows_pt)])


def _agg_call(y, zc, src_p, dst_p, n_acc, nchunk):
    c = y.shape[1]
    kfn = functools.partial(
        pl.kernel,
        out_type=jax.ShapeDtypeStruct((NC, n_acc, c), jnp.float32),
        mesh=_sc_mesh(),
        scratch_types=[
            pltpu.VMEM((nchunk, CH), jnp.int32),
            pltpu.VMEM((nchunk, CH), jnp.int32),
            pltpu.VMEM((CH, c), jnp.float32),
            pltpu.VMEM_SHARED((n_acc, c), jnp.float32),
            pltpu.SemaphoreType.DMA,
        ],
    )(_agg_kernel)
    return kfn(y, zc, src_p, dst_p)


def _dsqrt(degp):
    return lax.rsqrt(degp[0, :, 0:1] + degp[1, :, 0:1] + 1.0)


def _tc1_body(x_ref, w_ref, degp_ref, y_ref):
    d = _dsqrt(degp_ref[...])
    xw = jnp.dot(x_ref[...], w_ref[...], preferred_element_type=jnp.float32)
    y_ref[...] = xw * d


def _tc2_body(a_ref, degp_ref, w_ref, b_ref, y2_ref):
    d = _dsqrt(degp_ref[...])
    agg = a_ref[0] + a_ref[1]
    h = jnp.maximum(agg * d + b_ref[...], 0.0)
    y2_ref[...] = jnp.dot(h, w_ref[...], preferred_element_type=jnp.float32) * d


def _tc3_body(a_ref, degp_ref, b_ref, z_ref):
    d = _dsqrt(degp_ref[...])
    c_out = z_ref.shape[1]
    z_ref[...] = (a_ref[0, :, :c_out] + a_ref[1, :, :c_out]) * d + b_ref[...]


def kernel(x, edge_index, W1, b1, W2, b2):
    n = x.shape[0]
    e = edge_index.shape[1]
    c_in, c_hid = W1.shape
    c_out = W2.shape[1]
    n_acc = ((n + NS * 8) // (NS * 8)) * (NS * 8)
    epw = (((e + NW - 1) // NW) + CH - 1) // CH * CH
    e_pad = epw * NW
    nchunk = epw // CH

    src = edge_index[0].astype(jnp.int32)
    dst = edge_index[1].astype(jnp.int32)
    src_p = jnp.concatenate(
        [src, jnp.zeros((e_pad - e,), jnp.int32)]).reshape(NW, nchunk, CH)
    dst_p = jnp.concatenate(
        [dst, jnp.full((e_pad - e,), n, jnp.int32)]).reshape(NW, nchunk, CH)

    c_pad = max(c_out, 128)
    W2p = jnp.concatenate(
        [W2, jnp.zeros((c_hid, c_pad - c_out), jnp.float32)], axis=1)

    ones_w = jnp.ones((CH, DW), jnp.float32)
    z_deg = jnp.zeros((n_acc, DW), jnp.float32)
    z_hid = jnp.zeros((n_acc, c_hid), jnp.float32)
    b1r = b1.reshape(1, c_hid)
    b2r = b2.reshape(1, c_out)

    degp = _deg_call(dst_p, ones_w, z_deg, n_acc, nchunk)

    grid = (pl.cdiv(n_acc, BM),)
    y1 = pl.pallas_call(
        _tc1_body,
        grid=grid,
        in_specs=[
            pl.BlockSpec((BM, c_in), lambda i: (i, 0)),
            pl.BlockSpec((c_in, c_hid), lambda i: (0, 0)),
            pl.BlockSpec((NC, BM, DW), lambda i: (0, i, 0)),
        ],
        out_specs=pl.BlockSpec((BM, c_hid), lambda i: (i, 0)),
        out_shape=jax.ShapeDtypeStruct((n_acc, c_hid), jnp.float32),
    )(x, W1, degp)

    acc1 = _agg_call(y1, z_hid, src_p, dst_p, n_acc, nchunk)

    y2 = pl.pallas_call(
        _tc2_body,
        grid=grid,
        in_specs=[
            pl.BlockSpec((NC, BM, c_hid), lambda i: (0, i, 0)),
            pl.BlockSpec((NC, BM, DW), lambda i: (0, i, 0)),
            pl.BlockSpec((c_hid, c_pad), lambda i: (0, 0)),
            pl.BlockSpec((1, c_hid), lambda i: (0, 0)),
        ],
        out_specs=pl.BlockSpec((BM, c_pad), lambda i: (i, 0)),
        out_shape=jax.ShapeDtypeStruct((n_acc, c_pad), jnp.float32),
    )(acc1, degp, W2p, b1r)

    acc2 = _agg_call(y2, z_hid, src_p, dst_p, n_acc, nchunk)

    z = pl.pallas_call(
        _tc3_body,
        grid=grid,
        in_specs=[
            pl.BlockSpec((NC, BM, c_pad), lambda i: (0, i, 0)),
            pl.BlockSpec((NC, BM, DW), lambda i: (0, i, 0)),
            pl.BlockSpec((1, c_out), lambda i: (0, 0)),
        ],
        out_specs=pl.BlockSpec((BM, c_out), lambda i: (i, 0)),
        out_shape=jax.ShapeDtypeStruct((n, c_out), jnp.float32),
    )(acc2, degp, b2r)

    return z
